# baseline (speedup 1.0000x reference)
import functools

import jax
import jax.numpy as jnp
from jax import lax
from jax.experimental import pallas as pl
from jax.experimental.pallas import tpu as pltpu
from jax.experimental.pallas import tpu_sc as plsc

NUM_CORES = 2
NUM_SUBCORES = 16
NUM_WORKERS = NUM_CORES * NUM_SUBCORES
LANES = 16
NBUF = 6
CHUNK = 16


@functools.lru_cache(maxsize=None)
def _build(batch, seq, vocab, d_model):
    s_per_w = seq // NUM_WORKERS
    n_flat = batch * seq
    vregs_per_row = d_model // LANES
    n_groups = s_per_w // CHUNK
    n_chunks = n_groups * batch

    mesh = plsc.VectorSubcoreMesh(core_axis_name="c", subcore_axis_name="s")

    def chunk_bg(c):
        return c // batch, c % batch

    @functools.partial(
        pl.kernel,
        mesh=mesh,
        out_type=jax.ShapeDtypeStruct((n_flat, d_model), jnp.float32),
        scratch_types=[
            pltpu.VMEM((batch * s_per_w,), jnp.int32),
            pltpu.VMEM((s_per_w, d_model), jnp.float32),
        ]
        + [pltpu.VMEM((CHUNK, d_model), jnp.float32) for _ in range(NBUF)]
        + [pltpu.SemaphoreType.DMA for _ in range(2 * NBUF)],
    )
    def k(idx_hbm, emb_hbm, pos_hbm, out_hbm, idx_v, pos_v, *bufs):
        g = list(bufs[:NBUF])
        gsem = list(bufs[NBUF:2 * NBUF])
        wsem = list(bufs[2 * NBUF:3 * NBUF])
        wid = lax.axis_index("s") * NUM_CORES + lax.axis_index("c")
        s_base = wid * s_per_w

        idx_h = [pltpu.async_copy(idx_hbm.at[pl.ds(b * seq + s_base, s_per_w)],
                                  idx_v.at[pl.ds(b * s_per_w, s_per_w)],
                                  wsem[b])
                 for b in range(batch)]
        pos_h = pltpu.async_copy(pos_hbm.at[pl.ds(s_base, s_per_w)], pos_v,
                                 wsem[batch])
        for h in idx_h:
            h.wait()

        gh = [None] * NBUF
        wh = [None] * n_chunks

        def start_gather(c):
            hg, b = chunk_bg(c)
            i = c % NBUF
            gh[i] = pltpu.async_copy(
                emb_hbm.at[idx_v.at[pl.ds(b * s_per_w + hg * CHUNK, CHUNK)]],
                g[i], gsem[i])

        def wait_write(cn):
            if 0 <= cn < n_chunks and wh[cn] is not None:
                wh[cn].wait()
                wh[cn] = None

        for c in range(batch):
            start_gather(c)

        for hg in range(n_groups):
            c0 = hg * batch
            for b in range(batch):
                gh[(c0 + b) % NBUF].wait()
            if pos_h is not None:
                pos_h.wait()
                pos_h = None
            for c in (c0 + batch, c0 + batch + 1):
                if c < n_chunks:
                    wait_write(c - NBUF)
                    start_gather(c)

            gb = [g[(c0 + b) % NBUF] for b in range(batch)]

            def radd(r, gb=gb, hg=hg):
                for cc in range(vregs_per_row):
                    sl = pl.ds(cc * LANES, LANES)
                    pv = pos_v[hg * CHUNK + r, sl]
                    for b in range(batch):
                        plsc.addupdate(gb[b].at[r, sl], pv)

            plsc.parallel_loop(0, CHUNK, unroll=2)(radd)

            for b in range(batch):
                c = c0 + b
                row0 = b * seq + s_base + hg * CHUNK
                wh[c] = pltpu.async_copy(
                    g[c % NBUF], out_hbm.at[pl.ds(row0, CHUNK)],
                    wsem[c % NBUF])
            for c in (c0 + batch + 2, c0 + batch + 3):
                if c < n_chunks:
                    wait_write(c - NBUF)
                    start_gather(c)
        for c in range(n_chunks):
            wait_write(c)

    return k


def kernel(inputs, embeddings, position_embeddings):
    batch, seq = inputs.shape
    vocab, d_model = embeddings.shape
    idx_flat = inputs.reshape(-1).astype(jnp.int32)
    k = _build(batch, seq, vocab, d_model)
    out = k(idx_flat, embeddings, position_embeddings)
    return out.reshape(batch, seq, d_model)


# speedup vs baseline: 1.1641x; 1.1641x over previous
import functools

import jax
import jax.numpy as jnp
from jax import lax
from jax.experimental import pallas as pl
from jax.experimental.pallas import tpu as pltpu
from jax.experimental.pallas import tpu_sc as plsc

NUM_CORES = 2
NUM_SUBCORES = 16
NUM_WORKERS = NUM_CORES * NUM_SUBCORES
LANES = 16
NBUF = 6
CHUNK = 16


@functools.lru_cache(maxsize=None)
def _build(batch, seq, vocab, d_model):
    s_per_w = seq // NUM_WORKERS
    n_flat = batch * seq
    vregs_per_row = d_model // LANES
    n_groups = s_per_w // CHUNK
    n_chunks = n_groups * batch

    mesh = plsc.VectorSubcoreMesh(core_axis_name="c", subcore_axis_name="s")

    def chunk_bg(c):
        return c // batch, c % batch

    @functools.partial(
        pl.kernel,
        mesh=mesh,
        out_type=jax.ShapeDtypeStruct((n_flat, d_model), jnp.float32),
        scratch_types=[
            pltpu.VMEM((batch * s_per_w,), jnp.int32),
            pltpu.VMEM((s_per_w, d_model), jnp.float32),
        ]
        + [pltpu.VMEM((CHUNK, d_model), jnp.float32) for _ in range(NBUF)]
        + [pltpu.SemaphoreType.DMA for _ in range(2 * NBUF)],
    )
    def k(idx_hbm, emb_hbm, pos_hbm, out_hbm, idx_v, pos_v, *bufs):
        g = list(bufs[:NBUF])
        gsem = list(bufs[NBUF:2 * NBUF])
        wsem = list(bufs[2 * NBUF:3 * NBUF])
        wid = lax.axis_index("s") * NUM_CORES + lax.axis_index("c")
        s_base = wid * s_per_w

        idx_h = [pltpu.async_copy(idx_hbm.at[pl.ds(b * seq + s_base, s_per_w)],
                                  idx_v.at[pl.ds(b * s_per_w, s_per_w)],
                                  wsem[b])
                 for b in range(batch)]
        pos_h = pltpu.async_copy(pos_hbm.at[pl.ds(s_base, s_per_w)], pos_v,
                                 wsem[batch])
        for h in idx_h:
            h.wait()

        gh = [None] * NBUF
        wh = [None] * n_chunks

        def start_gather(c):
            hg, b = chunk_bg(c)
            i = c % NBUF
            gh[i] = pltpu.async_copy(
                emb_hbm.at[idx_v.at[pl.ds(b * s_per_w + hg * CHUNK, CHUNK)]],
                g[i], gsem[i])

        def wait_write(cn):
            if 0 <= cn < n_chunks and wh[cn] is not None:
                wh[cn].wait()
                wh[cn] = None

        for c in range(batch):
            start_gather(c)

        for hg in range(n_groups):
            c0 = hg * batch
            for b in range(batch):
                gh[(c0 + b) % NBUF].wait()
            if pos_h is not None:
                pos_h.wait()
                pos_h = None
            for c in (c0 + batch, c0 + batch + 1):
                if c < n_chunks:
                    wait_write(c - NBUF)
                    start_gather(c)

            gb = [g[(c0 + b) % NBUF] for b in range(batch)]

            def radd(r, _, gb=gb, hg=hg):
                for cc in range(vregs_per_row):
                    sl = pl.ds(cc * LANES, LANES)
                    pv = pos_v[hg * CHUNK + r, sl]
                    for b in range(batch):
                        plsc.addupdate(gb[b].at[r, sl], pv)
                return 0

            lax.fori_loop(0, CHUNK, radd, 0)

            for b in range(batch):
                c = c0 + b
                row0 = b * seq + s_base + hg * CHUNK
                wh[c] = pltpu.async_copy(
                    g[c % NBUF], out_hbm.at[pl.ds(row0, CHUNK)],
                    wsem[c % NBUF])
            for c in (c0 + batch + 2, c0 + batch + 3):
                if c < n_chunks:
                    wait_write(c - NBUF)
                    start_gather(c)
        for c in range(n_chunks):
            wait_write(c)

    return k


def kernel(inputs, embeddings, position_embeddings):
    batch, seq = inputs.shape
    vocab, d_model = embeddings.shape
    idx_flat = inputs.reshape(-1).astype(jnp.int32)
    k = _build(batch, seq, vocab, d_model)
    out = k(idx_flat, embeddings, position_embeddings)
    return out.reshape(batch, seq, d_model)
